# baseline (speedup 1.0000x reference)
import functools

import jax
import jax.numpy as jnp
from jax import lax
from jax.experimental import pallas as pl
from jax.experimental.pallas import tpu as pltpu
from jax.experimental.pallas import tpu_sc as plsc

B = 1024
D = 768
M = 32768
K = 32
KB = K * B
N = B * M

MC = 2048
MC2 = 1024

NC = 2
NS = 16
NW = NC * NS
LANES = 16

NPW = N // NW
H16 = 32768
HLO = 65536
CH16 = 32768
CHLO = 16384
STRIPE = 8192
UNROLL = 8
CAP = 6144
TGT_LOCAL = 48

_mesh = lambda: plsc.VectorSubcoreMesh(core_axis_name="c", subcore_axis_name="s")
_SC_PARAMS = pltpu.CompilerParams(needs_layout_passes=False)


def _enc_body(x_ref, ib_ref, w_ref, nb_ref, acts_ref):
    xc = x_ref[...] - ib_ref[...]
    pre = lax.dot_general(
        xc, w_ref[...], (((1,), (1,)), ((), ())),
        preferred_element_type=jnp.float32,
        precision=lax.Precision.DEFAULT,
    )
    acts_ref[...] = jnp.maximum(pre + nb_ref[...], 0.0)


def _encoder(x, W_enc, input_bias, neuron_bias):
    return pl.pallas_call(
        _enc_body,
        grid=(M // MC,),
        in_specs=[
            pl.BlockSpec((B, D), lambda i: (0, 0)),
            pl.BlockSpec((1, D), lambda i: (0, 0)),
            pl.BlockSpec((MC, D), lambda i: (i, 0)),
            pl.BlockSpec((1, MC), lambda i: (0, i)),
        ],
        out_specs=pl.BlockSpec((B, MC), lambda i: (0, i)),
        out_shape=jax.ShapeDtypeStruct((B, M), jnp.float32),
    )(x, input_bias.reshape(1, D), W_enc, neuron_bias.reshape(1, M))


def _zero_vmem(ref, n):
    def zbody(i, _):
        ref[pl.ds(i * LANES, LANES)] = jnp.zeros((LANES,), jnp.int32)
        return 0
    lax.fori_loop(0, n // LANES, zbody, 0)


def _merge_to_hbm(c, s, hist, shared, rbuf, acc, out_hbm, nbins, stripe):
    rb = stripe // NS
    for t in range(nbins // stripe):
        pltpu.sync_copy(hist.at[pl.ds(t * stripe, stripe)], shared.at[s])
        plsc.subcore_barrier()
        rbase = s * rb
        _zero_vmem(acc, rb)
        for k in range(NS):
            pltpu.sync_copy(shared.at[k, pl.ds(rbase, rb)], rbuf)

            def abody(i, _):
                sl = pl.ds(i * LANES, LANES)
                acc[sl] = acc[sl] + rbuf[sl]
                return 0
            lax.fori_loop(0, rb // LANES, abody, 0)
        pltpu.sync_copy(acc, out_hbm.at[c, pl.ds(t * stripe + rbase, rb)])
        plsc.subcore_barrier()


ROWS_PW = B // NW
HC = 16384
CPR = M // HC
NCHUNK = ROWS_PW * CPR


def _stream_rows(acts_hbm, row0, bufA, bufB, semA, semB, process):
    def addr(k):
        return acts_hbm.at[row0 + k // CPR, pl.ds((k % CPR) * HC, HC)]

    pltpu.async_copy(addr(0), bufA, semA)
    pltpu.async_copy(addr(1), bufB, semB)

    def pair(g, _):
        k = 2 * g
        pltpu.make_async_copy(addr(0), bufA, semA).wait()
        process(bufA)

        @pl.when(g < NCHUNK // 2 - 1)
        def _():
            pltpu.async_copy(addr(k + 2), bufA, semA)
        pltpu.make_async_copy(addr(0), bufB, semB).wait()
        process(bufB)

        @pl.when(g < NCHUNK // 2 - 1)
        def _():
            pltpu.async_copy(addr(k + 3), bufB, semB)
        return 0
    lax.fori_loop(0, NCHUNK // 2, pair, 0)


def _hist16_body(acts_hbm, out_hbm, bufA, bufB, hist, shared, rbuf, acc, semA, semB):
    c = lax.axis_index("c")
    s = lax.axis_index("s")
    wid = s * NC + c
    _zero_vmem(hist, H16)
    ones = jnp.ones((LANES,), jnp.int32)
    row0 = wid * ROWS_PW

    def process(buf):
        def vbody(i):
            v = buf[pl.ds(i * LANES, LANES)]
            m = v > 0.0
            b = plsc.bitcast(v, jnp.int32) >> 16
            plsc.addupdate_scatter(hist, [b], ones, mask=m)
        plsc.parallel_loop(0, HC // LANES, unroll=8)(vbody)

    _stream_rows(acts_hbm, row0, bufA, bufB, semA, semB, process)

    def tot_body(i, vacc):
        return vacc + hist[pl.ds(i * LANES, LANES)]
    vacc = lax.fori_loop(0, H16 // LANES, tot_body,
                         jnp.zeros((LANES,), jnp.int32))
    scattered = jnp.sum(vacc)
    lanes = lax.iota(jnp.int32, LANES)
    fix = jnp.where(lanes == 0, NPW - scattered, 0)
    hist[pl.ds(0, LANES)] = hist[pl.ds(0, LANES)] + fix

    _merge_to_hbm(c, s, hist, shared, rbuf, acc, out_hbm, H16, STRIPE)


def _hist16(acts):
    f = pl.kernel(
        _hist16_body,
        out_type=jax.ShapeDtypeStruct((NC, H16), jnp.int32),
        mesh=_mesh(),
        compiler_params=_SC_PARAMS,
        scratch_types=[
            pltpu.VMEM((HC,), jnp.float32),
            pltpu.VMEM((HC,), jnp.float32),
            pltpu.VMEM((H16,), jnp.int32),
            pltpu.VMEM_SHARED((NS, STRIPE), jnp.int32),
            pltpu.VMEM((STRIPE // NS,), jnp.int32),
            pltpu.VMEM((STRIPE // NS,), jnp.int32),
            pltpu.SemaphoreType.DMA,
            pltpu.SemaphoreType.DMA,
        ],
    )
    return f(acts)


def _histlo_body(acts_hbm, sc16_hbm, out_hbm, bufA, bufB, hist, shared, rbuf,
                 acc, svec, semA, semB):
    c = lax.axis_index("c")
    s = lax.axis_index("s")
    wid = s * NC + c
    pltpu.sync_copy(sc16_hbm, svec)
    bstar = svec[...][0]
    _zero_vmem(hist, HLO)
    ones = jnp.ones((LANES,), jnp.int32)
    row0 = wid * ROWS_PW

    def process(buf):
        def vbody(i):
            v = buf[pl.ds(i * LANES, LANES)]
            bits = plsc.bitcast(v, jnp.int32)
            hi = bits >> 16
            lo = bits & 0xFFFF
            m = hi == bstar
            plsc.addupdate_scatter(hist, [lo], ones, mask=m)
        plsc.parallel_loop(0, HC // LANES, unroll=8)(vbody)

    _stream_rows(acts_hbm, row0, bufA, bufB, semA, semB, process)
    _merge_to_hbm(c, s, hist, shared, rbuf, acc, out_hbm, HLO, STRIPE)


def _histlo(acts, sc16):
    f = pl.kernel(
        _histlo_body,
        out_type=jax.ShapeDtypeStruct((NC, HLO), jnp.int32),
        mesh=_mesh(),
        compiler_params=_SC_PARAMS,
        scratch_types=[
            pltpu.VMEM((HC,), jnp.float32),
            pltpu.VMEM((HC,), jnp.float32),
            pltpu.VMEM((HLO,), jnp.int32),
            pltpu.VMEM_SHARED((NS, STRIPE), jnp.int32),
            pltpu.VMEM((STRIPE // NS,), jnp.int32),
            pltpu.VMEM((STRIPE // NS,), jnp.int32),
            pltpu.VMEM((LANES,), jnp.int32),
            pltpu.SemaphoreType.DMA,
            pltpu.SemaphoreType.DMA,
        ],
    )
    return f(acts, sc16)


def _refine(v, cabove, target, lanes):
    rc = lax.rev(jnp.cumsum(lax.rev(v, (0,))), (0,))
    mask = (cabove + rc) >= target
    pc = plsc.all_reduce_population_count(mask)
    lstar = pc - 1
    sel = lanes == lstar
    v_l = jnp.sum(jnp.where(sel, v, 0))
    rc_l = jnp.sum(jnp.where(sel, rc, 0))
    l_sc = jnp.sum(jnp.where(sel, lanes, 0))
    return l_sc, cabove + rc_l - v_l


def _find_cross(read, nvec, target):
    def body(j2, st):
        carry, jstar, cabove = st
        j = nvec - 1 - j2
        bs = jnp.sum(read(j))
        newc = carry + bs
        cross = (carry < target) & (newc >= target)
        return (newc, jnp.where(cross, j, jstar),
                jnp.where(cross, carry, cabove))
    _, jstar, cabove = lax.fori_loop(
        0, nvec, body, (jnp.int32(0), jnp.int32(0), jnp.int32(0)))
    return jstar, cabove


def _hier_select(comb, sblk, sblk2, nbins, target):
    lanes = lax.iota(jnp.int32, LANES)
    nblk = nbins // LANES
    nblk2 = nblk // LANES

    def s1(j, _):
        base = j * (LANES * LANES)
        acc = jnp.zeros((LANES,), jnp.int32)
        for l in range(LANES):
            acc = acc + plsc.load_gather(comb, [base + lanes * LANES + l])
        sblk[pl.ds(j * LANES, LANES)] = acc
        return 0
    lax.fori_loop(0, nblk2, s1, 0)

    def s2(j, _):
        base = j * (LANES * LANES)
        acc = jnp.zeros((LANES,), jnp.int32)
        for l in range(LANES):
            acc = acc + plsc.load_gather(sblk, [base + lanes * LANES + l])
        sblk2[pl.ds(j * LANES, LANES)] = acc
        return 0
    lax.fori_loop(0, nblk2 // LANES, s2, 0)

    jv, cab = _find_cross(lambda j: sblk2[pl.ds(j * LANES, LANES)],
                          nblk2 // LANES, target)
    l2, cab = _refine(sblk2[pl.ds(jv * LANES, LANES)], cab, target, lanes)
    sb = jv * LANES + l2
    l1, cab = _refine(sblk[pl.ds(sb * LANES, LANES)], cab, target, lanes)
    blk = sb * LANES + l1
    l0, cab = _refine(comb[pl.ds(blk * LANES, LANES)], cab, target, lanes)
    return blk * LANES + l0, target - cab


def _load_combined(h2_hbm, comb, buf, nbins, chs):
    for r in range(NC):
        def lbody(i, _):
            pltpu.sync_copy(h2_hbm.at[r, pl.ds(i * chs, chs)], buf)

            def vb(j, _):
                slc = pl.ds(i * chs + j * LANES, LANES)
                slb = pl.ds(j * LANES, LANES)
                if r == 0:
                    comb[slc] = buf[slb]
                else:
                    comb[slc] = comb[slc] + buf[slb]
                return 0
            lax.fori_loop(0, chs // LANES, vb, 0)
            return 0
        lax.fori_loop(0, nbins // chs, lbody, 0)


def _scan16_body(h2_hbm, out_hbm, comb, buf, sblk, sblk2, ovec):
    c = lax.axis_index("c")
    s = lax.axis_index("s")
    wid = s * NC + c

    @pl.when(wid == 0)
    def _():
        _load_combined(h2_hbm, comb, buf, H16, 16384)
        bstar, need = _hier_select(comb, sblk, sblk2, H16, jnp.int32(KB))
        lanes = lax.iota(jnp.int32, LANES)
        outv = jnp.where(lanes == 0, jnp.full((LANES,), bstar, jnp.int32),
                         jnp.where(lanes == 1, jnp.full((LANES,), need, jnp.int32),
                                   jnp.zeros((LANES,), jnp.int32)))
        ovec[...] = outv
        pltpu.sync_copy(ovec, out_hbm)


def _scan16(h2):
    f = pl.kernel(
        _scan16_body,
        out_type=jax.ShapeDtypeStruct((LANES,), jnp.int32),
        mesh=_mesh(),
        compiler_params=_SC_PARAMS,
        scratch_types=[
            pltpu.VMEM((H16,), jnp.int32),
            pltpu.VMEM((16384,), jnp.int32),
            pltpu.VMEM((H16 // LANES,), jnp.int32),
            pltpu.VMEM((H16 // LANES // LANES,), jnp.int32),
            pltpu.VMEM((LANES,), jnp.int32),
        ],
    )
    return f(h2)


def _scanlo_body(h2_hbm, sc16_hbm, out_hbm, comb, buf, sblk, sblk2, svec, ovec):
    c = lax.axis_index("c")
    s = lax.axis_index("s")
    wid = s * NC + c

    @pl.when(wid == 0)
    def _():
        pltpu.sync_copy(sc16_hbm, svec)
        sv = svec[...]
        bstar = sv[0]
        need = sv[1]
        _load_combined(h2_hbm, comb, buf, HLO, 16384)
        tlo, _unused = _hier_select(comb, sblk, sblk2, HLO, need)
        tau_bits = (bstar << 16) | tlo
        tau_vec = plsc.bitcast(jnp.full((LANES,), tau_bits, jnp.int32),
                               jnp.float32)
        ovec[...] = tau_vec
        for k in range(8):
            pltpu.sync_copy(ovec, out_hbm.at[0, pl.ds(k * LANES, LANES)])


def _scanlo(h2, sc16):
    f = pl.kernel(
        _scanlo_body,
        out_type=jax.ShapeDtypeStruct((1, 128), jnp.float32),
        mesh=_mesh(),
        compiler_params=_SC_PARAMS,
        scratch_types=[
            pltpu.VMEM((HLO,), jnp.int32),
            pltpu.VMEM((16384,), jnp.int32),
            pltpu.VMEM((HLO // LANES,), jnp.int32),
            pltpu.VMEM((HLO // LANES // LANES,), jnp.int32),
            pltpu.VMEM((LANES,), jnp.int32),
            pltpu.VMEM((LANES,), jnp.float32),
        ],
    )
    return f(h2, sc16)


def _compact_body(acts_hbm, vals_hbm, info_hbm, bufA, bufB, hist, cbuf,
                  sblk, sblk2, posref, ivec, semA, semB):
    c = lax.axis_index("c")
    s = lax.axis_index("s")
    wid = s * NC + c
    row0 = wid * ROWS_PW
    lanes = lax.iota(jnp.int32, LANES)
    ones = jnp.ones((LANES,), jnp.int32)

    _zero_vmem(hist, H16)
    pltpu.sync_copy(acts_hbm.at[row0, pl.ds(0, HC)], bufA)

    def ph1(i, _):
        v = bufA[pl.ds(i * LANES, LANES)]
        m = v > 0.0
        b = plsc.bitcast(v, jnp.int32) >> 16
        plsc.addupdate_scatter(hist, [b], ones, mask=m)
        return 0
    lax.fori_loop(0, HC // LANES, ph1, 0)
    lw_bin, _r = _hier_select(hist, sblk, sblk2, H16, jnp.int32(TGT_LOCAL))
    lw_val = plsc.bitcast(jnp.full((LANES,), lw_bin << 16, jnp.int32),
                          jnp.float32)

    posref[...] = jnp.zeros((LANES,), jnp.int32)

    def process(buf):
        def vbody(i, pos):
            v = buf[pl.ds(i * LANES, LANES)]
            mc = v >= lw_val
            pc = plsc.all_reduce_population_count(mc)
            pos_sc = jnp.minimum(pos[0], CAP)
            plsc.store_compressed(cbuf.at[pl.ds(pos_sc, LANES)], v, mask=mc)
            return pos + pc
        pos0 = posref[...]
        posn = plsc.parallel_loop(0, HC // LANES, carry=pos0)(vbody)
        posref[...] = posn

    _stream_rows(acts_hbm, row0, bufA, bufB, semA, semB, process)

    pltpu.sync_copy(cbuf.at[pl.ds(0, CAP)], vals_hbm.at[wid])
    cnt = posref[...][0]
    ivec[...] = jnp.where(lanes == 0, jnp.full((LANES,), cnt, jnp.int32),
                          jnp.where(lanes == 1,
                                    jnp.full((LANES,), lw_bin, jnp.int32),
                                    jnp.zeros((LANES,), jnp.int32)))
    pltpu.sync_copy(ivec, info_hbm.at[wid])


def _compact(acts):
    f = pl.kernel(
        _compact_body,
        out_type=[
            jax.ShapeDtypeStruct((NW, CAP), jnp.float32),
            jax.ShapeDtypeStruct((NW, LANES), jnp.int32),
        ],
        mesh=_mesh(),
        compiler_params=_SC_PARAMS,
        scratch_types=[
            pltpu.VMEM((HC,), jnp.float32),
            pltpu.VMEM((HC,), jnp.float32),
            pltpu.VMEM((H16,), jnp.int32),
            pltpu.VMEM((CAP + LANES,), jnp.float32),
            pltpu.VMEM((H16 // LANES,), jnp.int32),
            pltpu.VMEM((H16 // LANES // LANES,), jnp.int32),
            pltpu.VMEM((LANES,), jnp.int32),
            pltpu.VMEM((LANES,), jnp.int32),
            pltpu.SemaphoreType.DMA,
            pltpu.SemaphoreType.DMA,
        ],
    )
    return f(acts)


def _select_body(vals_hbm, info_hbm, tau_hbm, flags_hbm, h16v, hlov, buf,
                 ibuf, sblk, sblk2, ovec, fvec):
    c = lax.axis_index("c")
    s = lax.axis_index("s")
    wid = s * NC + c

    @pl.when(wid == 0)
    def _():
        lanes = lax.iota(jnp.int32, LANES)
        ones = jnp.ones((LANES,), jnp.int32)
        pltpu.sync_copy(info_hbm, ibuf)
        total = jnp.int32(0)
        maxlw = jnp.int32(0)
        ovf = jnp.int32(0)
        for w in range(NW):
            iv = ibuf[w]
            cnt_w = iv[0]
            total = total + cnt_w
            maxlw = jnp.maximum(maxlw, iv[1])
            ovf = ovf | jnp.where(cnt_w > CAP, 1, 0)
        _zero_vmem(h16v, H16)
        _zero_vmem(hlov, HLO)
        for w in range(NW):
            pltpu.sync_copy(vals_hbm.at[w], buf)
            cnt_w = ibuf[w][0]

            def vA(i, _):
                v = buf[pl.ds(i * LANES, LANES)]
                m = (i * LANES + lanes) < cnt_w
                b = plsc.bitcast(v, jnp.int32) >> 16
                plsc.addupdate_scatter(h16v, [b], ones, mask=m)
                return 0
            lax.fori_loop(0, CAP // LANES, vA, 0)
        bstar, need = _hier_select(h16v, sblk, sblk2, H16, jnp.int32(KB))
        for w in range(NW):
            pltpu.sync_copy(vals_hbm.at[w], buf)
            cnt_w = ibuf[w][0]

            def vB(i, _):
                v = buf[pl.ds(i * LANES, LANES)]
                bits = plsc.bitcast(v, jnp.int32)
                m = ((i * LANES + lanes) < cnt_w) & ((bits >> 16) == bstar)
                plsc.addupdate_scatter(hlov, [bits & 0xFFFF], ones, mask=m)
                return 0
            lax.fori_loop(0, CAP // LANES, vB, 0)
        tlo, _unused = _hier_select(hlov, sblk, sblk2, HLO, need)
        tau_bits = (bstar << 16) | tlo
        ok = jnp.where((total >= KB) & (ovf == 0) & (maxlw <= bstar), 1, 0)
        ovec[...] = plsc.bitcast(jnp.full((LANES,), tau_bits, jnp.int32),
                                 jnp.float32)
        for k in range(8):
            pltpu.sync_copy(ovec, tau_hbm.at[0, pl.ds(k * LANES, LANES)])
        fvec[...] = jnp.full((LANES,), ok, jnp.int32)
        pltpu.sync_copy(fvec, flags_hbm)


def _select(vals, info):
    f = pl.kernel(
        _select_body,
        out_type=[
            jax.ShapeDtypeStruct((1, 128), jnp.float32),
            jax.ShapeDtypeStruct((LANES,), jnp.int32),
        ],
        mesh=_mesh(),
        compiler_params=_SC_PARAMS,
        scratch_types=[
            pltpu.VMEM((H16,), jnp.int32),
            pltpu.VMEM((HLO,), jnp.int32),
            pltpu.VMEM((CAP,), jnp.float32),
            pltpu.VMEM((NW, LANES), jnp.int32),
            pltpu.VMEM((HLO // LANES,), jnp.int32),
            pltpu.VMEM((HLO // LANES // LANES,), jnp.int32),
            pltpu.VMEM((LANES,), jnp.float32),
            pltpu.VMEM((LANES,), jnp.int32),
        ],
    )
    return f(vals, info)


def _dec_body(acts_ref, wd_ref, tau_ref, ib_ref, activ_ref, recon_ref, acc_ref):
    i = pl.program_id(0)

    @pl.when(i == 0)
    def _():
        acc_ref[...] = jnp.zeros_like(acc_ref)

    a = acts_ref[...]
    tau = tau_ref[0, 0]
    act = jnp.where(a >= tau, a, 0.0)
    activ_ref[...] = act
    acc_ref[...] += lax.dot_general(
        act, wd_ref[...], (((1,), (1,)), ((), ())),
        preferred_element_type=jnp.float32,
        precision=lax.Precision.DEFAULT,
    )

    @pl.when(i == M // MC2 - 1)
    def _():
        recon_ref[...] = acc_ref[...] + ib_ref[...]


def _decoder(acts, W_dec, tau, input_bias):
    return pl.pallas_call(
        _dec_body,
        grid=(M // MC2,),
        in_specs=[
            pl.BlockSpec((B, MC2), lambda i: (0, i)),
            pl.BlockSpec((D, MC2), lambda i: (0, i)),
            pl.BlockSpec((1, 128), lambda i: (0, 0)),
            pl.BlockSpec((1, D), lambda i: (0, 0)),
        ],
        out_specs=[
            pl.BlockSpec((B, MC2), lambda i: (0, i)),
            pl.BlockSpec((B, D), lambda i: (0, 0)),
        ],
        out_shape=[
            jax.ShapeDtypeStruct((B, M), jnp.float32),
            jax.ShapeDtypeStruct((B, D), jnp.float32),
        ],
        scratch_shapes=[pltpu.VMEM((B, D), jnp.float32)],
    )(acts, W_dec, tau, input_bias.reshape(1, D))


def kernel(x, W_enc, W_dec, input_bias, neuron_bias):
    acts = _encoder(x, W_enc, input_bias, neuron_bias)
    vals, info = _compact(acts)
    tau_fast, flags = _select(vals, info)

    def _fallback():
        h16 = _hist16(acts)
        sc16 = _scan16(h16)
        hlo = _histlo(acts, sc16)
        return _scanlo(hlo, sc16)

    tau = lax.cond(flags[0] == 1, lambda: tau_fast, _fallback)
    activ, recon = _decoder(acts, W_dec, tau, input_bias)
    return recon, activ


# speedup vs baseline: 2.2049x; 2.2049x over previous
import functools

import jax
import jax.numpy as jnp
from jax import lax
from jax.experimental import pallas as pl
from jax.experimental.pallas import tpu as pltpu
from jax.experimental.pallas import tpu_sc as plsc

B = 1024
D = 768
M = 32768
K = 32
KB = K * B
N = B * M

MC = 2048
MC2 = 1024

NC = 2
NS = 16
NW = NC * NS
LANES = 16

NPW = N // NW
H16 = 32768
HLO = 65536
CH16 = 32768
CHLO = 16384
STRIPE = 8192
UNROLL = 8
CAP = 6144
TGT_LOCAL = 48

_mesh = lambda: plsc.VectorSubcoreMesh(core_axis_name="c", subcore_axis_name="s")
_SC_PARAMS = pltpu.CompilerParams(needs_layout_passes=False)


def _enc_body(x_ref, ib_ref, w_ref, nb_ref, acts_ref):
    xc = x_ref[...] - ib_ref[...]
    pre = lax.dot_general(
        xc, w_ref[...], (((1,), (1,)), ((), ())),
        preferred_element_type=jnp.float32,
        precision=lax.Precision.DEFAULT,
    )
    acts_ref[...] = jnp.maximum(pre + nb_ref[...], 0.0)


def _encoder(x, W_enc, input_bias, neuron_bias):
    return pl.pallas_call(
        _enc_body,
        grid=(M // MC,),
        in_specs=[
            pl.BlockSpec((B, D), lambda i: (0, 0)),
            pl.BlockSpec((1, D), lambda i: (0, 0)),
            pl.BlockSpec((MC, D), lambda i: (i, 0)),
            pl.BlockSpec((1, MC), lambda i: (0, i)),
        ],
        out_specs=pl.BlockSpec((B, MC), lambda i: (0, i)),
        out_shape=jax.ShapeDtypeStruct((B, M), jnp.float32),
    )(x, input_bias.reshape(1, D), W_enc, neuron_bias.reshape(1, M))


def _zero_vmem(ref, n):
    def zbody(i):
        ref[pl.ds(i * LANES, LANES)] = jnp.zeros((LANES,), jnp.int32)
    plsc.parallel_loop(0, n // LANES, unroll=8)(zbody)


def _merge_to_hbm(c, s, hist, shared, rbuf, acc, out_hbm, nbins, stripe):
    rb = stripe // NS
    for t in range(nbins // stripe):
        pltpu.sync_copy(hist.at[pl.ds(t * stripe, stripe)], shared.at[s])
        plsc.subcore_barrier()
        rbase = s * rb
        _zero_vmem(acc, rb)
        for k in range(NS):
            pltpu.sync_copy(shared.at[k, pl.ds(rbase, rb)], rbuf)

            def abody(i, _):
                sl = pl.ds(i * LANES, LANES)
                acc[sl] = acc[sl] + rbuf[sl]
                return 0
            lax.fori_loop(0, rb // LANES, abody, 0)
        pltpu.sync_copy(acc, out_hbm.at[c, pl.ds(t * stripe + rbase, rb)])
        plsc.subcore_barrier()


ROWS_PW = B // NW
HC = 16384
CPR = M // HC
NCHUNK = ROWS_PW * CPR


def _stream_rows(acts_hbm, row0, bufA, bufB, semA, semB, process):
    def addr(k):
        return acts_hbm.at[row0 + k // CPR, pl.ds((k % CPR) * HC, HC)]

    pltpu.async_copy(addr(0), bufA, semA)
    pltpu.async_copy(addr(1), bufB, semB)

    def pair(g, _):
        k = 2 * g
        pltpu.make_async_copy(addr(0), bufA, semA).wait()
        process(bufA)

        @pl.when(g < NCHUNK // 2 - 1)
        def _():
            pltpu.async_copy(addr(k + 2), bufA, semA)
        pltpu.make_async_copy(addr(0), bufB, semB).wait()
        process(bufB)

        @pl.when(g < NCHUNK // 2 - 1)
        def _():
            pltpu.async_copy(addr(k + 3), bufB, semB)
        return 0
    lax.fori_loop(0, NCHUNK // 2, pair, 0)


def _hist16_body(acts_hbm, out_hbm, bufA, bufB, hist, shared, rbuf, acc, semA, semB):
    c = lax.axis_index("c")
    s = lax.axis_index("s")
    wid = s * NC + c
    _zero_vmem(hist, H16)
    ones = jnp.ones((LANES,), jnp.int32)
    row0 = wid * ROWS_PW

    def process(buf):
        def vbody(i):
            v = buf[pl.ds(i * LANES, LANES)]
            m = v > 0.0
            b = plsc.bitcast(v, jnp.int32) >> 16
            plsc.addupdate_scatter(hist, [b], ones, mask=m)
        plsc.parallel_loop(0, HC // LANES, unroll=8)(vbody)

    _stream_rows(acts_hbm, row0, bufA, bufB, semA, semB, process)

    def tot_body(i, vacc):
        return vacc + hist[pl.ds(i * LANES, LANES)]
    vacc = lax.fori_loop(0, H16 // LANES, tot_body,
                         jnp.zeros((LANES,), jnp.int32))
    scattered = jnp.sum(vacc)
    lanes = lax.iota(jnp.int32, LANES)
    fix = jnp.where(lanes == 0, NPW - scattered, 0)
    hist[pl.ds(0, LANES)] = hist[pl.ds(0, LANES)] + fix

    _merge_to_hbm(c, s, hist, shared, rbuf, acc, out_hbm, H16, STRIPE)


def _hist16(acts):
    f = pl.kernel(
        _hist16_body,
        out_type=jax.ShapeDtypeStruct((NC, H16), jnp.int32),
        mesh=_mesh(),
        compiler_params=_SC_PARAMS,
        scratch_types=[
            pltpu.VMEM((HC,), jnp.float32),
            pltpu.VMEM((HC,), jnp.float32),
            pltpu.VMEM((H16,), jnp.int32),
            pltpu.VMEM_SHARED((NS, STRIPE), jnp.int32),
            pltpu.VMEM((STRIPE // NS,), jnp.int32),
            pltpu.VMEM((STRIPE // NS,), jnp.int32),
            pltpu.SemaphoreType.DMA,
            pltpu.SemaphoreType.DMA,
        ],
    )
    return f(acts)


def _histlo_body(acts_hbm, sc16_hbm, out_hbm, bufA, bufB, hist, shared, rbuf,
                 acc, svec, semA, semB):
    c = lax.axis_index("c")
    s = lax.axis_index("s")
    wid = s * NC + c
    pltpu.sync_copy(sc16_hbm, svec)
    bstar = svec[...][0]
    _zero_vmem(hist, HLO)
    ones = jnp.ones((LANES,), jnp.int32)
    row0 = wid * ROWS_PW

    def process(buf):
        def vbody(i):
            v = buf[pl.ds(i * LANES, LANES)]
            bits = plsc.bitcast(v, jnp.int32)
            hi = bits >> 16
            lo = bits & 0xFFFF
            m = hi == bstar
            plsc.addupdate_scatter(hist, [lo], ones, mask=m)
        plsc.parallel_loop(0, HC // LANES, unroll=8)(vbody)

    _stream_rows(acts_hbm, row0, bufA, bufB, semA, semB, process)
    _merge_to_hbm(c, s, hist, shared, rbuf, acc, out_hbm, HLO, STRIPE)


def _histlo(acts, sc16):
    f = pl.kernel(
        _histlo_body,
        out_type=jax.ShapeDtypeStruct((NC, HLO), jnp.int32),
        mesh=_mesh(),
        compiler_params=_SC_PARAMS,
        scratch_types=[
            pltpu.VMEM((HC,), jnp.float32),
            pltpu.VMEM((HC,), jnp.float32),
            pltpu.VMEM((HLO,), jnp.int32),
            pltpu.VMEM_SHARED((NS, STRIPE), jnp.int32),
            pltpu.VMEM((STRIPE // NS,), jnp.int32),
            pltpu.VMEM((STRIPE // NS,), jnp.int32),
            pltpu.VMEM((LANES,), jnp.int32),
            pltpu.SemaphoreType.DMA,
            pltpu.SemaphoreType.DMA,
        ],
    )
    return f(acts, sc16)


def _refine(v, cabove, target, lanes):
    rc = lax.rev(jnp.cumsum(lax.rev(v, (0,))), (0,))
    mask = (cabove + rc) >= target
    pc = plsc.all_reduce_population_count(mask)
    lstar = pc - 1
    sel = lanes == lstar
    v_l = jnp.sum(jnp.where(sel, v, 0))
    rc_l = jnp.sum(jnp.where(sel, rc, 0))
    l_sc = jnp.sum(jnp.where(sel, lanes, 0))
    return l_sc, cabove + rc_l - v_l


def _find_cross(read, nvec, target):
    def body(j2, st):
        carry, jstar, cabove = st
        j = nvec - 1 - j2
        bs = jnp.sum(read(j))
        newc = carry + bs
        cross = (carry < target) & (newc >= target)
        return (newc, jnp.where(cross, j, jstar),
                jnp.where(cross, carry, cabove))
    _, jstar, cabove = lax.fori_loop(
        0, nvec, body, (jnp.int32(0), jnp.int32(0), jnp.int32(0)))
    return jstar, cabove


def _hier_select(comb, sblk, sblk2, nbins, target):
    lanes = lax.iota(jnp.int32, LANES)
    nblk = nbins // LANES
    nblk2 = nblk // LANES

    def s1(j, _):
        base = j * (LANES * LANES)
        acc = jnp.zeros((LANES,), jnp.int32)
        for l in range(LANES):
            acc = acc + plsc.load_gather(comb, [base + lanes * LANES + l])
        sblk[pl.ds(j * LANES, LANES)] = acc
        return 0
    lax.fori_loop(0, nblk2, s1, 0)

    def s2(j, _):
        base = j * (LANES * LANES)
        acc = jnp.zeros((LANES,), jnp.int32)
        for l in range(LANES):
            acc = acc + plsc.load_gather(sblk, [base + lanes * LANES + l])
        sblk2[pl.ds(j * LANES, LANES)] = acc
        return 0
    lax.fori_loop(0, nblk2 // LANES, s2, 0)

    jv, cab = _find_cross(lambda j: sblk2[pl.ds(j * LANES, LANES)],
                          nblk2 // LANES, target)
    l2, cab = _refine(sblk2[pl.ds(jv * LANES, LANES)], cab, target, lanes)
    sb = jv * LANES + l2
    l1, cab = _refine(sblk[pl.ds(sb * LANES, LANES)], cab, target, lanes)
    blk = sb * LANES + l1
    l0, cab = _refine(comb[pl.ds(blk * LANES, LANES)], cab, target, lanes)
    return blk * LANES + l0, target - cab


def _load_combined(h2_hbm, comb, buf, nbins, chs):
    for r in range(NC):
        def lbody(i, _):
            pltpu.sync_copy(h2_hbm.at[r, pl.ds(i * chs, chs)], buf)

            def vb(j, _):
                slc = pl.ds(i * chs + j * LANES, LANES)
                slb = pl.ds(j * LANES, LANES)
                if r == 0:
                    comb[slc] = buf[slb]
                else:
                    comb[slc] = comb[slc] + buf[slb]
                return 0
            lax.fori_loop(0, chs // LANES, vb, 0)
            return 0
        lax.fori_loop(0, nbins // chs, lbody, 0)


def _scan16_body(h2_hbm, out_hbm, comb, buf, sblk, sblk2, ovec):
    c = lax.axis_index("c")
    s = lax.axis_index("s")
    wid = s * NC + c

    @pl.when(wid == 0)
    def _():
        _load_combined(h2_hbm, comb, buf, H16, 16384)
        bstar, need = _hier_select(comb, sblk, sblk2, H16, jnp.int32(KB))
        lanes = lax.iota(jnp.int32, LANES)
        outv = jnp.where(lanes == 0, jnp.full((LANES,), bstar, jnp.int32),
                         jnp.where(lanes == 1, jnp.full((LANES,), need, jnp.int32),
                                   jnp.zeros((LANES,), jnp.int32)))
        ovec[...] = outv
        pltpu.sync_copy(ovec, out_hbm)


def _scan16(h2):
    f = pl.kernel(
        _scan16_body,
        out_type=jax.ShapeDtypeStruct((LANES,), jnp.int32),
        mesh=_mesh(),
        compiler_params=_SC_PARAMS,
        scratch_types=[
            pltpu.VMEM((H16,), jnp.int32),
            pltpu.VMEM((16384,), jnp.int32),
            pltpu.VMEM((H16 // LANES,), jnp.int32),
            pltpu.VMEM((H16 // LANES // LANES,), jnp.int32),
            pltpu.VMEM((LANES,), jnp.int32),
        ],
    )
    return f(h2)


def _scanlo_body(h2_hbm, sc16_hbm, out_hbm, comb, buf, sblk, sblk2, svec, ovec):
    c = lax.axis_index("c")
    s = lax.axis_index("s")
    wid = s * NC + c

    @pl.when(wid == 0)
    def _():
        pltpu.sync_copy(sc16_hbm, svec)
        sv = svec[...]
        bstar = sv[0]
        need = sv[1]
        _load_combined(h2_hbm, comb, buf, HLO, 16384)
        tlo, _unused = _hier_select(comb, sblk, sblk2, HLO, need)
        tau_bits = (bstar << 16) | tlo
        tau_vec = plsc.bitcast(jnp.full((LANES,), tau_bits, jnp.int32),
                               jnp.float32)
        ovec[...] = tau_vec
        for k in range(8):
            pltpu.sync_copy(ovec, out_hbm.at[0, pl.ds(k * LANES, LANES)])


def _scanlo(h2, sc16):
    f = pl.kernel(
        _scanlo_body,
        out_type=jax.ShapeDtypeStruct((1, 128), jnp.float32),
        mesh=_mesh(),
        compiler_params=_SC_PARAMS,
        scratch_types=[
            pltpu.VMEM((HLO,), jnp.int32),
            pltpu.VMEM((16384,), jnp.int32),
            pltpu.VMEM((HLO // LANES,), jnp.int32),
            pltpu.VMEM((HLO // LANES // LANES,), jnp.int32),
            pltpu.VMEM((LANES,), jnp.int32),
            pltpu.VMEM((LANES,), jnp.float32),
        ],
    )
    return f(h2, sc16)


def _compact_body(acts_hbm, vals_hbm, info_hbm, bufA, bufB, hist, cbuf,
                  sblk, sblk2, posref, ivec, semA, semB):
    c = lax.axis_index("c")
    s = lax.axis_index("s")
    wid = s * NC + c
    row0 = wid * ROWS_PW
    lanes = lax.iota(jnp.int32, LANES)
    ones = jnp.ones((LANES,), jnp.int32)

    _zero_vmem(hist, H16)
    pltpu.sync_copy(acts_hbm.at[row0, pl.ds(0, HC)], bufA)

    def ph1(i):
        v = bufA[pl.ds(i * LANES, LANES)]
        m = v > 0.0
        b = plsc.bitcast(v, jnp.int32) >> 16
        plsc.addupdate_scatter(hist, [b], ones, mask=m)
    plsc.parallel_loop(0, HC // LANES, unroll=8)(ph1)
    lw_bin, _r = _hier_select(hist, sblk, sblk2, H16, jnp.int32(TGT_LOCAL))
    lw_val = plsc.bitcast(jnp.full((LANES,), lw_bin << 16, jnp.int32),
                          jnp.float32)

    posref[...] = jnp.zeros((LANES,), jnp.int32)

    capv = jnp.full((LANES,), CAP + LANES - 1, jnp.int32)

    def process(buf):
        def vbody(i, pos):
            v = buf[pl.ds(i * LANES, LANES)]
            mc = v >= lw_val
            pc = plsc.all_reduce_population_count(mc)
            ci = jnp.cumsum(jnp.where(mc, 1, 0))
            idx = jnp.minimum(pos + ci - 1, capv)
            plsc.store_scatter(cbuf, [idx], v, mask=mc)
            return pos + pc
        pos0 = posref[...]
        posn = plsc.parallel_loop(0, HC // LANES, unroll=8, carry=pos0)(vbody)
        posref[...] = posn

    _stream_rows(acts_hbm, row0, bufA, bufB, semA, semB, process)

    pltpu.sync_copy(cbuf.at[pl.ds(0, CAP)], vals_hbm.at[wid])
    cnt = posref[...][0]
    ivec[...] = jnp.where(lanes == 0, jnp.full((LANES,), cnt, jnp.int32),
                          jnp.where(lanes == 1,
                                    jnp.full((LANES,), lw_bin, jnp.int32),
                                    jnp.zeros((LANES,), jnp.int32)))
    pltpu.sync_copy(ivec, info_hbm.at[wid])


def _compact(acts):
    f = pl.kernel(
        _compact_body,
        out_type=[
            jax.ShapeDtypeStruct((NW, CAP), jnp.float32),
            jax.ShapeDtypeStruct((NW, LANES), jnp.int32),
        ],
        mesh=_mesh(),
        compiler_params=_SC_PARAMS,
        scratch_types=[
            pltpu.VMEM((HC,), jnp.float32),
            pltpu.VMEM((HC,), jnp.float32),
            pltpu.VMEM((H16,), jnp.int32),
            pltpu.VMEM((CAP + LANES,), jnp.float32),
            pltpu.VMEM((H16 // LANES,), jnp.int32),
            pltpu.VMEM((H16 // LANES // LANES,), jnp.int32),
            pltpu.VMEM((LANES,), jnp.int32),
            pltpu.VMEM((LANES,), jnp.int32),
            pltpu.SemaphoreType.DMA,
            pltpu.SemaphoreType.DMA,
        ],
    )
    return f(acts)


SBATCH = 4


def _select_body(vals_hbm, info_hbm, tau_hbm, flags_hbm, hv, bufA, bufB,
                 ibuf, sblk, sblk2, ovec, fvec, semA, semB):
    c = lax.axis_index("c")
    s = lax.axis_index("s")
    wid = s * NC + c

    @pl.when(wid == 0)
    def _():
        lanes = lax.iota(jnp.int32, LANES)
        ones = jnp.ones((LANES,), jnp.int32)
        pltpu.sync_copy(info_hbm, ibuf)
        total = jnp.int32(0)
        maxlw = jnp.int32(0)
        ovf = jnp.int32(0)
        for w in range(NW):
            iv = ibuf[w]
            cnt_w = iv[0]
            total = total + cnt_w
            maxlw = jnp.maximum(maxlw, iv[1])
            ovf = ovf | jnp.where(cnt_w > CAP, 1, 0)

        nb = NW // SBATCH

        def sweep(scatter_fn):
            def bat(g):
                return vals_hbm.at[pl.ds(g * SBATCH, SBATCH)]
            pltpu.async_copy(bat(0), bufA, semA)
            for g in range(nb):
                buf = bufA if g % 2 == 0 else bufB
                sem = semA if g % 2 == 0 else semB
                nbuf = bufB if g % 2 == 0 else bufA
                nsem = semB if g % 2 == 0 else semA
                if g + 1 < nb:
                    pltpu.async_copy(bat(g + 1), nbuf, nsem)
                pltpu.make_async_copy(bat(0), buf, sem).wait()
                for ww in range(SBATCH):
                    w = g * SBATCH + ww
                    cnt_w = ibuf[w][0]

                    def vb(i):
                        v = buf[ww, pl.ds(i * LANES, LANES)]
                        m = (i * LANES + lanes) < cnt_w
                        scatter_fn(v, m)
                    plsc.parallel_loop(0, CAP // LANES, unroll=8)(vb)

        _zero_vmem(hv, HLO)

        def scatA(v, m):
            b = plsc.bitcast(v, jnp.int32) >> 16
            plsc.addupdate_scatter(hv, [b], ones, mask=m)
        sweep(scatA)
        bstar, need = _hier_select(hv, sblk, sblk2, H16, jnp.int32(KB))

        _zero_vmem(hv, HLO)

        def scatB(v, m):
            bits = plsc.bitcast(v, jnp.int32)
            m2 = m & ((bits >> 16) == bstar)
            plsc.addupdate_scatter(hv, [bits & 0xFFFF], ones, mask=m2)
        sweep(scatB)
        tlo, _unused = _hier_select(hv, sblk, sblk2, HLO, need)

        tau_bits = (bstar << 16) | tlo
        ok = jnp.where((total >= KB) & (ovf == 0) & (maxlw <= bstar), 1, 0)
        ovec[...] = plsc.bitcast(jnp.full((LANES,), tau_bits, jnp.int32),
                                 jnp.float32)
        for k in range(8):
            pltpu.sync_copy(ovec, tau_hbm.at[0, pl.ds(k * LANES, LANES)])
        fvec[...] = jnp.full((LANES,), ok, jnp.int32)
        pltpu.sync_copy(fvec, flags_hbm)


def _select(vals, info):
    f = pl.kernel(
        _select_body,
        out_type=[
            jax.ShapeDtypeStruct((1, 128), jnp.float32),
            jax.ShapeDtypeStruct((LANES,), jnp.int32),
        ],
        mesh=_mesh(),
        compiler_params=_SC_PARAMS,
        scratch_types=[
            pltpu.VMEM((HLO,), jnp.int32),
            pltpu.VMEM((SBATCH, CAP), jnp.float32),
            pltpu.VMEM((SBATCH, CAP), jnp.float32),
            pltpu.VMEM((NW, LANES), jnp.int32),
            pltpu.VMEM((HLO // LANES,), jnp.int32),
            pltpu.VMEM((HLO // LANES // LANES,), jnp.int32),
            pltpu.VMEM((LANES,), jnp.float32),
            pltpu.VMEM((LANES,), jnp.int32),
            pltpu.SemaphoreType.DMA,
            pltpu.SemaphoreType.DMA,
        ],
    )
    return f(vals, info)


def _dec_body(acts_ref, wd_ref, tau_ref, ib_ref, activ_ref, recon_ref, acc_ref):
    i = pl.program_id(0)

    @pl.when(i == 0)
    def _():
        acc_ref[...] = jnp.zeros_like(acc_ref)

    a = acts_ref[...]
    tau = tau_ref[0, 0]
    act = jnp.where(a >= tau, a, 0.0)
    activ_ref[...] = act
    acc_ref[...] += lax.dot_general(
        act, wd_ref[...], (((1,), (1,)), ((), ())),
        preferred_element_type=jnp.float32,
        precision=lax.Precision.DEFAULT,
    )

    @pl.when(i == M // MC2 - 1)
    def _():
        recon_ref[...] = acc_ref[...] + ib_ref[...]


def _decoder(acts, W_dec, tau, input_bias):
    return pl.pallas_call(
        _dec_body,
        grid=(M // MC2,),
        in_specs=[
            pl.BlockSpec((B, MC2), lambda i: (0, i)),
            pl.BlockSpec((D, MC2), lambda i: (0, i)),
            pl.BlockSpec((1, 128), lambda i: (0, 0)),
            pl.BlockSpec((1, D), lambda i: (0, 0)),
        ],
        out_specs=[
            pl.BlockSpec((B, MC2), lambda i: (0, i)),
            pl.BlockSpec((B, D), lambda i: (0, 0)),
        ],
        out_shape=[
            jax.ShapeDtypeStruct((B, M), jnp.float32),
            jax.ShapeDtypeStruct((B, D), jnp.float32),
        ],
        scratch_shapes=[pltpu.VMEM((B, D), jnp.float32)],
    )(acts, W_dec, tau, input_bias.reshape(1, D))


def kernel(x, W_enc, W_dec, input_bias, neuron_bias):
    acts = _encoder(x, W_enc, input_bias, neuron_bias)
    vals, info = _compact(acts)
    tau_fast, flags = _select(vals, info)

    def _fallback():
        h16 = _hist16(acts)
        sc16 = _scan16(h16)
        hlo = _histlo(acts, sc16)
        return _scanlo(hlo, sc16)

    tau = lax.cond(flags[0] == 1, lambda: tau_fast, _fallback)
    activ, recon = _decoder(acts, W_dec, tau, input_bias)
    return recon, activ
